# baseline (speedup 1.0000x reference)
import functools

import jax
import jax.numpy as jnp
from jax import lax
from jax.experimental import pallas as pl
from jax.experimental.pallas import tpu as pltpu

_INTERPRET = False

INF = float("inf")
NEG_INF = float("-inf")


def _fps_body(n, px_ref, py_ref, pz_ref, qx_ref, qy_ref, qz_ref, dists_ref):
    R = px_ref.shape[0]
    M = R * 128
    row = lax.broadcasted_iota(jnp.int32, (R, 128), 0)
    col = lax.broadcasted_iota(jnp.int32, (R, 128), 1)
    iota = row * 128 + col
    px = px_ref[...]
    py = py_ref[...]
    pz = pz_ref[...]
    dists_ref[...] = jnp.full((R, 128), INF, jnp.float32)

    def body(i, j):
        onehot = iota == j
        lx = jnp.sum(jnp.where(onehot, px, 0.0), keepdims=True)
        ly = jnp.sum(jnp.where(onehot, py, 0.0), keepdims=True)
        lz = jnp.sum(jnp.where(onehot, pz, 0.0), keepdims=True)
        qx_ref[pl.ds(i - 1, 1), :] = lx
        qy_ref[pl.ds(i - 1, 1), :] = ly
        qz_ref[pl.ds(i - 1, 1), :] = lz
        dx = px - lx
        dy = py - ly
        dz = pz - lz
        d = dx * dx + dy * dy + dz * dz
        nd = jnp.minimum(dists_ref[...], d)
        dists_ref[...] = nd
        m = jnp.max(nd, keepdims=True)
        j2 = jnp.min(jnp.where(nd == m, iota, jnp.int32(M)), keepdims=True)
        return j2

    j0 = jnp.zeros((1, 1), jnp.int32)
    jlast = lax.fori_loop(1, n, body, j0)
    onehot = iota == jlast
    qx_ref[pl.ds(n - 1, 1), :] = jnp.sum(jnp.where(onehot, px, 0.0), keepdims=True)
    qy_ref[pl.ds(n - 1, 1), :] = jnp.sum(jnp.where(onehot, py, 0.0), keepdims=True)
    qz_ref[pl.ds(n - 1, 1), :] = jnp.sum(jnp.where(onehot, pz, 0.0), keepdims=True)


def _fps_call(px, py, pz, n):
    R = px.shape[0]
    out = pl.pallas_call(
        functools.partial(_fps_body, n),
        out_shape=[jax.ShapeDtypeStruct((n, 1), jnp.float32)] * 3,
        scratch_shapes=[pltpu.VMEM((R, 128), jnp.float32)],
        interpret=_INTERPRET,
    )(px, py, pz)
    return out


def _sa1_body(r2, k, CB, W, px_ref, py_ref, pz_ref, qx_ref, qy_ref, qz_ref,
              w0_ref, b0_ref, w1_ref, b1_ref, w2_ref, b2_ref,
              out_ref, d2_ref, bv_ref, bi_ref, bx_ref, by_ref, bz_ref):
    M = px_ref.shape[1]
    G = M // W
    px = px_ref[...]
    py = py_ref[...]
    pz = pz_ref[...]
    qx = qx_ref[...]
    qy = qy_ref[...]
    qz = qz_ref[...]

    qq = qx * qx + qy * qy + qz * qz
    pp = px * px + py * py + pz * pz
    qmat = jnp.concatenate([qx, qy, qz], axis=1)
    pmat = jnp.concatenate([px, py, pz], axis=0)
    cross = jnp.dot(qmat, pmat, preferred_element_type=jnp.float32)
    d2 = qq + pp - 2.0 * cross
    d2 = jnp.maximum(d2, 0.0)
    d2 = jnp.where(d2 <= r2, d2, INF)
    d2_ref[...] = d2

    iota_full = lax.broadcasted_iota(jnp.int32, (CB, M), 1)
    lane_w = lax.broadcasted_iota(jnp.int32, (1, W), 1)

    def rebuild(d2x):
        bv = jnp.full((CB, W), INF, jnp.float32)
        bi = jnp.full((CB, W), -1, jnp.int32)
        bx = jnp.zeros((CB, W), jnp.float32)
        by = jnp.zeros((CB, W), jnp.float32)
        bz = jnp.zeros((CB, W), jnp.float32)
        s2v, s2i, s2x, s2y, s2z = bv, bi, bx, by, bz
        s3v, s3i, s3x, s3y, s3z = bv, bi, bx, by, bz
        for s in range(G):
            ev = d2x[:, s * W:(s + 1) * W]
            ei = lane_w + jnp.int32(s * W)
            ex = px[:, s * W:(s + 1) * W]
            ey = py[:, s * W:(s + 1) * W]
            ez = pz[:, s * W:(s + 1) * W]
            lt1 = (ev < bv) | ((ev == bv) & (ei < bi))
            lt2 = (ev < s2v) | ((ev == s2v) & (ei < s2i))
            lt3 = (ev < s3v) | ((ev == s3v) & (ei < s3i))
            n3v = jnp.where(lt2, s2v, jnp.where(lt3, ev, s3v))
            n3i = jnp.where(lt2, s2i, jnp.where(lt3, ei, s3i))
            n3x = jnp.where(lt2, s2x, jnp.where(lt3, ex, s3x))
            n3y = jnp.where(lt2, s2y, jnp.where(lt3, ey, s3y))
            n3z = jnp.where(lt2, s2z, jnp.where(lt3, ez, s3z))
            n2v = jnp.where(lt1, bv, jnp.where(lt2, ev, s2v))
            n2i = jnp.where(lt1, bi, jnp.where(lt2, ei, s2i))
            n2x = jnp.where(lt1, bx, jnp.where(lt2, ex, s2x))
            n2y = jnp.where(lt1, by, jnp.where(lt2, ey, s2y))
            n2z = jnp.where(lt1, bz, jnp.where(lt2, ez, s2z))
            bv = jnp.where(lt1, ev, bv)
            bi = jnp.where(lt1, ei, bi)
            bx = jnp.where(lt1, ex, bx)
            by = jnp.where(lt1, ey, by)
            bz = jnp.where(lt1, ez, bz)
            s2v, s2i, s2x, s2y, s2z = n2v, n2i, n2x, n2y, n2z
            s3v, s3i, s3x, s3y, s3z = n3v, n3i, n3x, n3y, n3z
        bv_ref[...] = jnp.concatenate([bv, s2v, s3v], axis=0)
        bi_ref[...] = jnp.concatenate([bi, s2i, s3i], axis=0)
        bx_ref[...] = jnp.concatenate([bx, s2x, s3x], axis=0)
        by_ref[...] = jnp.concatenate([by, s2y, s3y], axis=0)
        bz_ref[...] = jnp.concatenate([bz, s2z, s3z], axis=0)

    rebuild(d2)

    mvals, sxs, sys_, szs, js = [], [], [], [], []
    for t in range(k):
        bv = bv_ref[0:CB, :]
        bi = bi_ref[0:CB, :]
        m = jnp.min(bv, axis=1, keepdims=True)
        jstar = jnp.min(jnp.where(bv == m, bi, jnp.int32(M)),
                        axis=1, keepdims=True)
        selmask = (bv == m) & (bi == jstar)
        sxs.append(jnp.sum(jnp.where(selmask, bx_ref[0:CB, :], 0.0),
                           axis=1, keepdims=True))
        sys_.append(jnp.sum(jnp.where(selmask, by_ref[0:CB, :], 0.0),
                            axis=1, keepdims=True))
        szs.append(jnp.sum(jnp.where(selmask, bz_ref[0:CB, :], 0.0),
                           axis=1, keepdims=True))
        mvals.append(m)
        js.append(jstar)
        s2i_old = bi_ref[CB:2 * CB, :]
        promoted_i = jnp.sum(jnp.where(selmask, s2i_old, 0),
                             axis=1, keepdims=True)
        for ref in (bv_ref, bi_ref, bx_ref, by_ref, bz_ref):
            cur = ref[...]
            b, s2, s3 = cur[0:CB, :], cur[CB:2 * CB, :], cur[2 * CB:3 * CB, :]
            sent = (jnp.full((CB, W), INF, cur.dtype) if ref is bv_ref
                    else jnp.full((CB, W), -1, cur.dtype) if ref is bi_ref
                    else jnp.zeros((CB, W), cur.dtype))
            ref[...] = jnp.concatenate([
                jnp.where(selmask, s2, b),
                jnp.where(selmask, s3, s2),
                jnp.where(selmask, sent, s3)], axis=0)
        need = jnp.min(promoted_i) < 0

        @pl.when(need)
        def _():
            d2x = d2_ref[...]
            for jprev in js:
                d2x = jnp.where(iota_full == jprev, INF, d2x)
            rebuild(d2x)

    mv = jnp.concatenate(mvals, axis=0)
    qxk = jnp.concatenate([qx] * k, axis=0)
    qyk = jnp.concatenate([qy] * k, axis=0)
    qzk = jnp.concatenate([qz] * k, axis=0)
    relx = jnp.concatenate(sxs, axis=0) - qxk
    rely = jnp.concatenate(sys_, axis=0) - qyk
    relz = jnp.concatenate(szs, axis=0) - qzk

    rel = jnp.concatenate([relx, rely, relz], axis=1)
    h = jnp.dot(rel, w0_ref[...], preferred_element_type=jnp.float32) + b0_ref[...]
    h1 = jnp.maximum(h, 0.0)
    h2 = jnp.maximum(
        jnp.dot(h1, w1_ref[...], preferred_element_type=jnp.float32) + b1_ref[...], 0.0)
    h3 = jnp.maximum(
        jnp.dot(h2, w2_ref[...], preferred_element_type=jnp.float32) + b2_ref[...], 0.0)
    h3 = jnp.where(mv < INF, h3, NEG_INF)
    out_ref[...] = jnp.max(h3.reshape(k, CB, 128), axis=0)


def _sa1_call(pxr, pyr, pzr, qx, qy, qz, w0, b0, w1, b1, w2, b2, r, k, CB, W):
    M = pxr.shape[1]
    ncent = qx.shape[0]
    grid = ncent // CB
    fixed = lambda i: (0, 0)
    return pl.pallas_call(
        functools.partial(_sa1_body, r * r, k, CB, W),
        grid=(grid,),
        in_specs=[
            pl.BlockSpec((1, M), fixed),
            pl.BlockSpec((1, M), fixed),
            pl.BlockSpec((1, M), fixed),
            pl.BlockSpec((CB, 1), lambda i: (i, 0)),
            pl.BlockSpec((CB, 1), lambda i: (i, 0)),
            pl.BlockSpec((CB, 1), lambda i: (i, 0)),
            pl.BlockSpec(w0.shape, fixed),
            pl.BlockSpec(b0.shape, fixed),
            pl.BlockSpec(w1.shape, fixed),
            pl.BlockSpec(b1.shape, fixed),
            pl.BlockSpec(w2.shape, fixed),
            pl.BlockSpec(b2.shape, fixed),
        ],
        out_specs=pl.BlockSpec((CB, 128), lambda i: (i, 0)),
        out_shape=jax.ShapeDtypeStruct((ncent, 128), jnp.float32),
        scratch_shapes=[pltpu.VMEM((CB, M), jnp.float32),
                        pltpu.VMEM((3 * CB, W), jnp.float32),
                        pltpu.VMEM((3 * CB, W), jnp.int32),
                        pltpu.VMEM((3 * CB, W), jnp.float32),
                        pltpu.VMEM((3 * CB, W), jnp.float32),
                        pltpu.VMEM((3 * CB, W), jnp.float32)],
        interpret=_INTERPRET,
    )(pxr, pyr, pzr, qx, qy, qz, w0, b0, w1, b1, w2, b2)


def _sa2_body(r2, k, CB, px_ref, py_ref, pz_ref, qx_ref, qy_ref, qz_ref,
              x1_ref, w0a_ref, w0b_ref, b0_ref, w1_ref, b1_ref, w2_ref, b2_ref,
              out_ref):
    M = px_ref.shape[1]
    px = px_ref[...]
    py = py_ref[...]
    pz = pz_ref[...]
    qx = qx_ref[...]
    qy = qy_ref[...]
    qz = qz_ref[...]

    qq = qx * qx + qy * qy + qz * qz
    pp = px * px + py * py + pz * pz
    qmat = jnp.concatenate([qx, qy, qz], axis=1)
    pmat = jnp.concatenate([px, py, pz], axis=0)
    cross = jnp.dot(qmat, pmat, preferred_element_type=jnp.float32)
    d2 = qq + pp - 2.0 * cross
    d2 = jnp.maximum(d2, 0.0)
    d2 = jnp.where(d2 <= r2, d2, INF)

    iota = lax.broadcasted_iota(jnp.int32, (CB, M), 1)
    mvals, sxs, sys_, szs, onehots = [], [], [], [], []
    for _ in range(k):
        m = jnp.min(d2, axis=1, keepdims=True)
        sel = d2 == m
        idxs = jnp.min(jnp.where(sel, iota, jnp.int32(M)), axis=1, keepdims=True)
        exact = iota == idxs
        sxs.append(jnp.sum(jnp.where(exact, px, 0.0), axis=1, keepdims=True))
        sys_.append(jnp.sum(jnp.where(exact, py, 0.0), axis=1, keepdims=True))
        szs.append(jnp.sum(jnp.where(exact, pz, 0.0), axis=1, keepdims=True))
        mvals.append(m)
        onehots.append(jnp.where(exact, 1.0, 0.0))
        d2 = jnp.where(exact, INF, d2)

    mv = jnp.concatenate(mvals, axis=0)
    qxk = jnp.concatenate([qx] * k, axis=0)
    qyk = jnp.concatenate([qy] * k, axis=0)
    qzk = jnp.concatenate([qz] * k, axis=0)
    relx = jnp.concatenate(sxs, axis=0) - qxk
    rely = jnp.concatenate(sys_, axis=0) - qyk
    relz = jnp.concatenate(szs, axis=0) - qzk

    O = jnp.concatenate(onehots, axis=0)
    xg = jnp.dot(O, x1_ref[...], preferred_element_type=jnp.float32,
                 precision=lax.Precision.HIGHEST)

    rel = jnp.concatenate([relx, rely, relz], axis=1)
    ha = jnp.dot(xg, w0a_ref[...], preferred_element_type=jnp.float32)
    hb = jnp.dot(rel, w0b_ref[...], preferred_element_type=jnp.float32)
    h1 = jnp.maximum(ha + hb + b0_ref[...], 0.0)
    h2 = jnp.maximum(
        jnp.dot(h1, w1_ref[...], preferred_element_type=jnp.float32) + b1_ref[...], 0.0)
    h3 = jnp.maximum(
        jnp.dot(h2, w2_ref[...], preferred_element_type=jnp.float32) + b2_ref[...], 0.0)
    h3 = jnp.where(mv < INF, h3, NEG_INF)
    out_ref[...] = jnp.max(h3.reshape(k, CB, 256), axis=0)


def _sa2_call(pxr, pyr, pzr, qx, qy, qz, x1, w0a, w0b, b0, w1, b1, w2, b2,
              r, k, CB):
    M = pxr.shape[1]
    ncent = qx.shape[0]
    grid = ncent // CB
    fixed = lambda i: (0, 0)
    return pl.pallas_call(
        functools.partial(_sa2_body, r * r, k, CB),
        grid=(grid,),
        in_specs=[
            pl.BlockSpec((1, M), fixed),
            pl.BlockSpec((1, M), fixed),
            pl.BlockSpec((1, M), fixed),
            pl.BlockSpec((CB, 1), lambda i: (i, 0)),
            pl.BlockSpec((CB, 1), lambda i: (i, 0)),
            pl.BlockSpec((CB, 1), lambda i: (i, 0)),
            pl.BlockSpec(x1.shape, fixed),
            pl.BlockSpec(w0a.shape, fixed),
            pl.BlockSpec(w0b.shape, fixed),
            pl.BlockSpec(b0.shape, fixed),
            pl.BlockSpec(w1.shape, fixed),
            pl.BlockSpec(b1.shape, fixed),
            pl.BlockSpec(w2.shape, fixed),
            pl.BlockSpec(b2.shape, fixed),
        ],
        out_specs=pl.BlockSpec((CB, 256), lambda i: (i, 0)),
        out_shape=jax.ShapeDtypeStruct((ncent, 256), jnp.float32),
        compiler_params=pltpu.CompilerParams(
            dimension_semantics=("parallel",)),
        interpret=_INTERPRET,
    )(pxr, pyr, pzr, qx, qy, qz, x1, w0a, w0b, b0, w1, b1, w2, b2)


def _tail_body(x2_ref, qx_ref, qy_ref, qz_ref,
               w0a_ref, w0b_ref, b0_ref, w1_ref, b1_ref, w2_ref, b2_ref,
               hw0_ref, hb0_ref, hw1_ref, hb1_ref, hw2_ref, hb2_ref,
               out_ref):
    qx = qx_ref[...]
    qy = qy_ref[...]
    qz = qz_ref[...]
    dx = qx - jnp.mean(qx, keepdims=True)
    dy = qy - jnp.mean(qy, keepdims=True)
    dz = qz - jnp.mean(qz, keepdims=True)

    rel = jnp.concatenate([dx, dy, dz], axis=1)
    h1 = (jnp.dot(x2_ref[...], w0a_ref[...], preferred_element_type=jnp.float32)
          + jnp.dot(rel, w0b_ref[...], preferred_element_type=jnp.float32)
          + b0_ref[...])
    h1 = jnp.maximum(h1, 0.0)
    h2 = jnp.maximum(
        jnp.dot(h1, w1_ref[...], preferred_element_type=jnp.float32) + b1_ref[...], 0.0)
    h3 = jnp.maximum(
        jnp.dot(h2, w2_ref[...], preferred_element_type=jnp.float32) + b2_ref[...], 0.0)
    g = jnp.max(h3, axis=0, keepdims=True)
    o1 = jnp.maximum(
        jnp.dot(g, hw0_ref[...], preferred_element_type=jnp.float32) + hb0_ref[...], 0.0)
    o2 = jnp.maximum(
        jnp.dot(o1, hw1_ref[...], preferred_element_type=jnp.float32) + hb1_ref[...], 0.0)
    out_ref[...] = (
        jnp.dot(o2, hw2_ref[...], preferred_element_type=jnp.float32) + hb2_ref[...])


def _tail_call(x2, qx, qy, qz, w0a, w0b, b0, w1, b1, w2, b2,
               hw0, hb0, hw1, hb1, hw2, hb2):
    return pl.pallas_call(
        _tail_body,
        out_shape=jax.ShapeDtypeStruct((1, 40), jnp.float32),
        interpret=_INTERPRET,
    )(x2, qx, qy, qz, w0a, w0b, b0, w1, b1, w2, b2,
      hw0, hb0, hw1, hb1, hw2, hb2)


def kernel(pos, sa1_w0, sa1_b0, sa1_w1, sa1_b1, sa1_w2, sa1_b2,
           sa2_w0, sa2_b0, sa2_w1, sa2_b1, sa2_w2, sa2_b2,
           sa3_w0, sa3_b0, sa3_w1, sa3_b1, sa3_w2, sa3_b2,
           head_w0, head_b0, head_w1, head_b1, head_w2, head_b2):
    B, N, _ = pos.shape
    p = pos.reshape(B * N, 3)
    M1 = B * N
    px = p[:, 0]
    py = p[:, 1]
    pz = p[:, 2]

    qx1, qy1, qz1 = _fps_call(px.reshape(M1 // 128, 128),
                              py.reshape(M1 // 128, 128),
                              pz.reshape(M1 // 128, 128), 512)
    x1 = _sa1_call(px.reshape(1, M1), py.reshape(1, M1), pz.reshape(1, M1),
                   qx1, qy1, qz1,
                   sa1_w0, sa1_b0.reshape(1, 64),
                   sa1_w1, sa1_b1.reshape(1, 64),
                   sa1_w2, sa1_b2.reshape(1, 128),
                   r=0.2, k=32, CB=32, W=2048)

    qx2, qy2, qz2 = _fps_call(qx1.reshape(4, 128), qy1.reshape(4, 128),
                              qz1.reshape(4, 128), 128)
    x2 = _sa2_call(qx1.reshape(1, 512), qy1.reshape(1, 512), qz1.reshape(1, 512),
                   qx2, qy2, qz2, x1,
                   sa2_w0[:128], sa2_w0[128:], sa2_b0.reshape(1, 128),
                   sa2_w1, sa2_b1.reshape(1, 128),
                   sa2_w2, sa2_b2.reshape(1, 256),
                   r=0.4, k=64, CB=16)

    out = _tail_call(x2, qx2, qy2, qz2,
                     sa3_w0[:256], sa3_w0[256:], sa3_b0.reshape(1, 256),
                     sa3_w1, sa3_b1.reshape(1, 512),
                     sa3_w2, sa3_b2.reshape(1, 1024),
                     head_w0, head_b0.reshape(1, 512),
                     head_w1, head_b1.reshape(1, 256),
                     head_w2, head_b2.reshape(1, 40))
    return out
